# Initial kernel scaffold; baseline (speedup 1.0000x reference)
import jax
import jax.numpy as jnp
from jax.experimental import pallas as pl


def kernel(x, edge_index, W1, b1, W2, b2, W3, b3, W4, b4, Wl, bl):
    raise NotImplementedError("write your pallas kernel here")


# speedup vs baseline: 9.5946x; 9.5946x over previous
import functools

import jax
import jax.numpy as jnp
from jax import lax
from jax.experimental import pallas as pl
from jax.experimental.pallas import tpu as pltpu
from jax.experimental.pallas import tpu_sc as plsc

N = 10000
E = 320000
F = 128
C = 64
NC, NS = 2, 16
NW = NC * NS
CHUNK = 128
NCH = 79
EPT = NCH * CHUNK
EPAD = EPT * NW
NPAD = 10240
RPT = NPAD // NS
BLK = 1024
OBLK = 1000


def _mesh():
    return plsc.VectorSubcoreMesh(core_axis_name="c", subcore_axis_name="s")


@functools.partial(
    pl.kernel,
    out_type=jax.ShapeDtypeStruct((NC, NPAD, 16), jnp.float32),
    mesh=_mesh(),
    scratch_types=[
        pltpu.VMEM((NCH, CHUNK), jnp.int32),
        pltpu.VMEM((CHUNK, 16), jnp.float32),
        pltpu.VMEM_SHARED((NPAD, 16), jnp.float32),
    ],
)
def _deg_kernel(dstp_hbm, e1_hbm, z16_hbm, out_hbm, dst_v, e1_v, acc):
    cid = lax.axis_index("c")
    sid = lax.axis_index("s")
    wid = sid * NC + cid
    pltpu.sync_copy(z16_hbm, acc.at[pl.ds(sid * RPT, RPT)])
    pltpu.sync_copy(dstp_hbm.at[wid], dst_v)
    pltpu.sync_copy(e1_hbm, e1_v)
    plsc.subcore_barrier()

    def body(j, carry):
        pltpu.sync_copy(e1_v, acc.at[dst_v.at[j]], add=True)
        return carry

    lax.fori_loop(0, NCH, body, 0)
    plsc.subcore_barrier()
    pltpu.sync_copy(acc.at[pl.ds(sid * RPT, RPT)],
                    out_hbm.at[cid, pl.ds(sid * RPT, RPT)])


@functools.partial(
    pl.kernel,
    out_type=jax.ShapeDtypeStruct((NC, NPAD, F), jnp.float32),
    mesh=_mesh(),
    scratch_types=[
        pltpu.VMEM((NCH, CHUNK), jnp.int32),
        pltpu.VMEM((NCH, CHUNK), jnp.int32),
        pltpu.VMEM((CHUNK, F), jnp.float32),
        pltpu.VMEM_SHARED((NPAD, F), jnp.float32),
        pltpu.SemaphoreType.DMA,
    ],
)
def _agg_kernel(hp_hbm, srcp_hbm, dstp_hbm, z128_hbm, out_hbm,
                src_v, dst_v, rows_v, acc, sem):
    cid = lax.axis_index("c")
    sid = lax.axis_index("s")
    wid = sid * NC + cid
    pltpu.sync_copy(z128_hbm, acc.at[pl.ds(sid * RPT, RPT)])
    pltpu.sync_copy(srcp_hbm.at[wid], src_v)
    pltpu.sync_copy(dstp_hbm.at[wid], dst_v)
    plsc.subcore_barrier()

    def body(j, carry):
        pltpu.async_copy(hp_hbm.at[src_v.at[j]], rows_v, sem).wait()
        pltpu.sync_copy(rows_v, acc.at[dst_v.at[j]], add=True)
        return carry

    lax.fori_loop(0, NCH, body, 0)
    plsc.subcore_barrier()
    pltpu.sync_copy(acc.at[pl.ds(sid * RPT, RPT)],
                    out_hbm.at[cid, pl.ds(sid * RPT, RPT)])


def _dis(degs_ref):
    deg = degs_ref[0, :, 0] + degs_ref[1, :, 0] + 1.0
    return lax.rsqrt(deg)[:, None]


def _mm_scale_body(x_ref, w_ref, degs_ref, o_ref):
    h = jnp.dot(x_ref[...], w_ref[...], preferred_element_type=jnp.float32)
    o_ref[...] = h * _dis(degs_ref)


def _combine_mm_body(p_ref, hp_ref, degs_ref, b_ref, w_ref, o_ref, *, act):
    dis = _dis(degs_ref)
    xn = dis * (p_ref[0] + p_ref[1] + hp_ref[...]) + b_ref[...]
    if act:
        xn = jnp.maximum(xn, 0.0)
    o_ref[...] = dis * jnp.dot(xn, w_ref[...],
                               preferred_element_type=jnp.float32)


def _head_body(p_ref, hp_ref, degs_ref, b_ref, wl_ref, bl_ref, o_ref):
    dis = _dis(degs_ref)
    x4 = dis * (p_ref[0] + p_ref[1] + hp_ref[...]) + b_ref[...]
    logits = jnp.dot(x4, wl_ref[...], preferred_element_type=jnp.float32)
    logits = logits + bl_ref[...]
    m = jnp.max(logits, axis=1, keepdims=True)
    shifted = logits - m
    o_ref[...] = shifted - jnp.log(
        jnp.sum(jnp.exp(shifted), axis=1, keepdims=True))


def _mm_scale(x, w, degs):
    grid = NPAD // BLK
    return pl.pallas_call(
        _mm_scale_body,
        grid=(grid,),
        in_specs=[
            pl.BlockSpec((BLK, F), lambda i: (i, 0)),
            pl.BlockSpec((F, F), lambda i: (0, 0)),
            pl.BlockSpec((NC, BLK, 16), lambda i: (0, i, 0)),
        ],
        out_specs=pl.BlockSpec((BLK, F), lambda i: (i, 0)),
        out_shape=jax.ShapeDtypeStruct((NPAD, F), jnp.float32),
    )(x, w, degs)


def _combine_mm(p, hp, degs, b, w, act):
    grid = NPAD // BLK
    return pl.pallas_call(
        functools.partial(_combine_mm_body, act=act),
        grid=(grid,),
        in_specs=[
            pl.BlockSpec((NC, BLK, F), lambda i: (0, i, 0)),
            pl.BlockSpec((BLK, F), lambda i: (i, 0)),
            pl.BlockSpec((NC, BLK, 16), lambda i: (0, i, 0)),
            pl.BlockSpec((1, F), lambda i: (0, 0)),
            pl.BlockSpec((F, F), lambda i: (0, 0)),
        ],
        out_specs=pl.BlockSpec((BLK, F), lambda i: (i, 0)),
        out_shape=jax.ShapeDtypeStruct((NPAD, F), jnp.float32),
    )(p, hp, degs, b.reshape(1, F), w)


def _head(p, hp, degs, b, wl, bl):
    grid = N // OBLK
    return pl.pallas_call(
        _head_body,
        grid=(grid,),
        in_specs=[
            pl.BlockSpec((NC, OBLK, F), lambda i: (0, i, 0)),
            pl.BlockSpec((OBLK, F), lambda i: (i, 0)),
            pl.BlockSpec((NC, OBLK, 16), lambda i: (0, i, 0)),
            pl.BlockSpec((1, F), lambda i: (0, 0)),
            pl.BlockSpec((F, C), lambda i: (0, 0)),
            pl.BlockSpec((1, C), lambda i: (0, 0)),
        ],
        out_specs=pl.BlockSpec((OBLK, C), lambda i: (i, 0)),
        out_shape=jax.ShapeDtypeStruct((N, C), jnp.float32),
    )(p, hp, degs, b.reshape(1, F), wl, bl.reshape(1, C))


def kernel(x, edge_index, W1, b1, W2, b2, W3, b3, W4, b4, Wl, bl):
    ei = edge_index.astype(jnp.int32)
    pad = jnp.full((EPAD - E,), N, jnp.int32)
    srcp = jnp.concatenate([ei[0], pad]).reshape(NW, NCH, CHUNK)
    dstp = jnp.concatenate([ei[1], pad]).reshape(NW, NCH, CHUNK)
    e1 = jnp.zeros((CHUNK, 16), jnp.float32).at[:, 0].set(1.0)
    z16 = jnp.zeros((RPT, 16), jnp.float32)
    z128 = jnp.zeros((RPT, F), jnp.float32)

    degs = _deg_kernel(dstp, e1, z16)
    h1 = _mm_scale(x, W1, degs)
    p = _agg_kernel(h1, srcp, dstp, z128)
    h2 = _combine_mm(p, h1, degs, b1, W2, act=True)
    p = _agg_kernel(h2, srcp, dstp, z128)
    h3 = _combine_mm(p, h2, degs, b2, W3, act=False)
    p = _agg_kernel(h3, srcp, dstp, z128)
    h4 = _combine_mm(p, h3, degs, b3, W4, act=True)
    p = _agg_kernel(h4, srcp, dstp, z128)
    return _head(p, h4, degs, b4, Wl, bl)
